# Initial kernel scaffold; baseline (speedup 1.0000x reference)
import functools

import jax
import jax.numpy as jnp
from jax import lax
from jax.experimental import pallas as pl
from jax.experimental.pallas import tpu as pltpu
from jax.experimental.pallas import tpu_sc as plsc

_SAMPLE_NUM = 1152
_EPS = 1e-5

_info = plsc.get_sparse_core_info()
_NC, _NS = _info.num_cores, _info.num_subcores
_NW = _NC * _NS


def _sc_gather_add(total0, total1, C):
    r0 = total0 // _NW
    r1 = total1 // _NW
    mesh = plsc.VectorSubcoreMesh(core_axis_name="c", subcore_axis_name="s")

    @functools.partial(
        pl.kernel,
        mesh=mesh,
        out_type=(
            jax.ShapeDtypeStruct((total0, C), jnp.float32),
            jax.ShapeDtypeStruct((total1, C), jnp.float32),
        ),
        scratch_types=[
            pltpu.VMEM((r0,), jnp.int32),
            pltpu.VMEM((r0, C), jnp.float32),
            pltpu.VMEM((r0, C), jnp.float32),
            pltpu.VMEM((r1,), jnp.int32),
            pltpu.VMEM((r1,), jnp.int32),
            pltpu.VMEM((r1, C), jnp.float32),
            pltpu.VMEM((r1, C), jnp.float32),
            pltpu.SemaphoreType.DMA,
        ],
    )
    def k(xg_hbm, pg_hbm, xa_hbm, idx_hbm, posd_hbm, pe_hbm,
          out0_hbm, out1_hbm,
          pg_v, g_x, g_pf, idx_v, posd_v, a_x, a_pf, sem):
        wid = lax.axis_index("s") * _NC + lax.axis_index("c")

        b0 = wid * r0
        pltpu.sync_copy(pg_hbm.at[pl.ds(b0, r0)], pg_v)
        cx = pltpu.async_copy(xg_hbm.at[pl.ds(b0, r0)], g_x, sem)
        cp = pltpu.async_copy(pe_hbm.at[pg_v], g_pf, sem)
        cx.wait()
        cp.wait()
        for i in range(0, r0 * C, 16):
            r, c = i // C, i % C
            g_x[r, pl.ds(c, 16)] = g_x[r, pl.ds(c, 16)] + g_pf[r, pl.ds(c, 16)]
        pltpu.sync_copy(g_x, out0_hbm.at[pl.ds(b0, r0)])

        b1 = wid * r1
        pltpu.sync_copy(idx_hbm.at[pl.ds(b1, r1)], idx_v)
        pltpu.sync_copy(posd_hbm.at[pl.ds(b1, r1)], posd_v)
        ca = pltpu.async_copy(xa_hbm.at[idx_v], a_x, sem)
        cb = pltpu.async_copy(pe_hbm.at[posd_v], a_pf, sem)
        ca.wait()
        cb.wait()
        for i in range(0, r1 * C, 16):
            r, c = i // C, i % C
            a_x[r, pl.ds(c, 16)] = a_x[r, pl.ds(c, 16)] + a_pf[r, pl.ds(c, 16)]
        pltpu.sync_copy(a_x, out1_hbm.at[pl.ds(b1, r1)])

    return k


def kernel(x_grid, x_ada, pos_grid, pos_ada, pos_embed, norm_w, norm_b,
           conf_w, conf_b):
    B, N_g, C = x_grid.shape
    N = x_ada.shape[1]
    S = _SAMPLE_NUM

    mu = jnp.mean(x_ada, axis=-1, keepdims=True)
    var = jnp.var(x_ada, axis=-1, keepdims=True)
    normed = (x_ada - mu) / jnp.sqrt(var + _EPS) * norm_w + norm_b
    conf = normed @ conf_w + conf_b
    conf = jax.nn.softmax(conf, axis=1) * N
    _, idx = jax.lax.top_k(conf[..., 0], S)

    pos_down = jnp.take_along_axis(pos_ada, idx, axis=1)

    idx_g = (idx + jnp.arange(B, dtype=idx.dtype)[:, None] * N).reshape(-1)
    posd_f = pos_down.reshape(-1).astype(jnp.int32)
    pg_f = pos_grid.reshape(-1).astype(jnp.int32)
    pe = pos_embed[0]

    sc = _sc_gather_add(B * N_g, B * S, C)
    out0_f, out1_f = sc(
        x_grid.reshape(B * N_g, C), pg_f,
        x_ada.reshape(B * N, C), idx_g.astype(jnp.int32), posd_f, pe)
    out0 = out0_f.reshape(B, N_g, C)
    out1 = out1_f.reshape(B, S, C)
    return out0, out1, pos_grid, pos_ada


# speedup vs baseline: 1.2116x; 1.2116x over previous
import functools

import jax
import jax.numpy as jnp
from jax import lax
from jax.experimental import pallas as pl
from jax.experimental.pallas import tpu as pltpu
from jax.experimental.pallas import tpu_sc as plsc

_SAMPLE_NUM = 1152
_EPS = 1e-5

_info = plsc.get_sparse_core_info()
_NC, _NS = _info.num_cores, _info.num_subcores
_NW = _NC * _NS


def _sc_gather_add(total0, total1, C):
    r0 = total0 // _NW
    r1 = total1 // _NW
    CH = 48
    n_ch = r1 // CH
    mesh = plsc.VectorSubcoreMesh(core_axis_name="c", subcore_axis_name="s")

    @functools.partial(
        pl.kernel,
        mesh=mesh,
        out_type=(
            jax.ShapeDtypeStruct((total0, C), jnp.float32),
            jax.ShapeDtypeStruct((total1, C), jnp.float32),
        ),
        scratch_types=[
            pltpu.VMEM((r0,), jnp.int32),
            pltpu.VMEM((r0, C), jnp.float32),
            pltpu.VMEM((r0, C), jnp.float32),
            pltpu.VMEM((CH,), jnp.int32),
            pltpu.VMEM((CH,), jnp.int32),
            pltpu.VMEM((CH, C), jnp.float32),
            pltpu.VMEM((CH, C), jnp.float32),
            pltpu.SemaphoreType.DMA,
        ],
    )
    def k(xg_hbm, pg_hbm, xa_hbm, idx_hbm, posd_hbm, pe_hbm,
          out0_hbm, out1_hbm,
          pg_v, g_x, g_pf, idx_v, posd_v, a_x, a_pf, sem):
        wid = lax.axis_index("s") * _NC + lax.axis_index("c")

        b0 = wid * r0
        pltpu.sync_copy(pg_hbm.at[pl.ds(b0, r0)], pg_v)
        cx = pltpu.async_copy(xg_hbm.at[pl.ds(b0, r0)], g_x, sem)
        cp = pltpu.async_copy(pe_hbm.at[pg_v], g_pf, sem)
        cx.wait()
        cp.wait()

        def add_row0(r, carry):
            for c in range(0, C, 16):
                g_x[r, pl.ds(c, 16)] = (g_x[r, pl.ds(c, 16)]
                                        + g_pf[r, pl.ds(c, 16)])
            return carry

        lax.fori_loop(0, r0, add_row0, 0)
        pltpu.sync_copy(g_x, out0_hbm.at[pl.ds(b0, r0)])

        def add_row1(r, carry):
            for c in range(0, C, 16):
                a_x[r, pl.ds(c, 16)] = (a_x[r, pl.ds(c, 16)]
                                        + a_pf[r, pl.ds(c, 16)])
            return carry

        for k in range(n_ch):
            b1 = wid * r1 + k * CH
            pltpu.sync_copy(idx_hbm.at[pl.ds(b1, CH)], idx_v)
            pltpu.sync_copy(posd_hbm.at[pl.ds(b1, CH)], posd_v)
            ca = pltpu.async_copy(xa_hbm.at[idx_v], a_x, sem)
            cb = pltpu.async_copy(pe_hbm.at[posd_v], a_pf, sem)
            ca.wait()
            cb.wait()
            lax.fori_loop(0, CH, add_row1, 0)
            pltpu.sync_copy(a_x, out1_hbm.at[pl.ds(b1, CH)])

    return k


def kernel(x_grid, x_ada, pos_grid, pos_ada, pos_embed, norm_w, norm_b,
           conf_w, conf_b):
    B, N_g, C = x_grid.shape
    N = x_ada.shape[1]
    S = _SAMPLE_NUM

    mu = jnp.mean(x_ada, axis=-1, keepdims=True)
    var = jnp.var(x_ada, axis=-1, keepdims=True)
    normed = (x_ada - mu) / jnp.sqrt(var + _EPS) * norm_w + norm_b
    conf = normed @ conf_w + conf_b
    conf = jax.nn.softmax(conf, axis=1) * N
    _, idx = jax.lax.top_k(conf[..., 0], S)

    pos_down = jnp.take_along_axis(pos_ada, idx, axis=1)

    idx_g = (idx + jnp.arange(B, dtype=idx.dtype)[:, None] * N).reshape(-1)
    posd_f = pos_down.reshape(-1).astype(jnp.int32)
    pg_f = pos_grid.reshape(-1).astype(jnp.int32)
    pe = pos_embed[0]

    sc = _sc_gather_add(B * N_g, B * S, C)
    out0_f, out1_f = sc(
        x_grid.reshape(B * N_g, C), pg_f,
        x_ada.reshape(B * N, C), idx_g.astype(jnp.int32), posd_f, pe)
    out0 = out0_f.reshape(B, N_g, C)
    out1 = out1_f.reshape(B, S, C)
    return out0, out1, pos_grid, pos_ada
